# Initial kernel scaffold; baseline (speedup 1.0000x reference)
import jax
import jax.numpy as jnp
from jax.experimental import pallas as pl


def kernel(x, edge_index, W1_l, b1_l, W1_r, W2_l, b2_l, W2_r):
    raise NotImplementedError("write your pallas kernel here")


# speedup vs baseline: 4.6700x; 4.6700x over previous
import jax
import jax.numpy as jnp
from jax import lax
from jax.experimental import pallas as pl
from jax.experimental.pallas import tpu as pltpu
from jax.experimental.pallas import tpu_sc as plsc

N = 10000
E = 320000
D = 128
NC, NS = 2, 16
NW = NC * NS
EPT = E // NW
CH = 80
NCHUNK = EPT // CH
NP = 10240
RPT = NP // NS

_MESH = plsc.VectorSubcoreMesh(core_axis_name="c", subcore_axis_name="s",
                               num_cores=NC, num_subcores=NS)


def _zero_rows(buf, rows):
  def zrow(i, carry):
    for j in range(D // 16):
      buf[i, pl.ds(j * 16, 16)] = jnp.zeros((16,), jnp.float32)
    return carry
  lax.fori_loop(0, rows, zrow, 0)


@pl.kernel(
    out_type=[jax.ShapeDtypeStruct((NC * NP, D), jnp.float32)],
    mesh=_MESH,
    scratch_types=[
        pltpu.VMEM((CH,), jnp.int32),
        pltpu.VMEM((CH,), jnp.int32),
        pltpu.VMEM((CH, D), jnp.float32),
        pltpu.VMEM_SHARED((NP, D), jnp.float32),
        pltpu.SemaphoreType.DMA,
    ])
def _seg_sum(tbl, src, dst, out, src_v, dst_v, rows_v, acc, sem):
  cid = lax.axis_index("c")
  sid = lax.axis_index("s")
  wid = cid * NS + sid

  _zero_rows(rows_v, CH)
  r0 = sid * RPT
  for k in range(RPT // CH):
    pltpu.sync_copy(rows_v, acc.at[pl.ds(r0 + k * CH, CH)])
  plsc.subcore_barrier()

  ebase = wid * EPT
  def chunk(c, carry):
    base = pl.multiple_of(ebase + c * CH, 8)
    pltpu.sync_copy(src.at[pl.ds(base, CH)], src_v)
    pltpu.sync_copy(dst.at[pl.ds(base, CH)], dst_v)
    pltpu.async_copy(tbl.at[src_v], rows_v, sem).wait()
    pltpu.sync_copy(rows_v, acc.at[dst_v], add=True)
    return carry
  lax.fori_loop(0, NCHUNK, chunk, 0)
  plsc.subcore_barrier()

  pltpu.sync_copy(acc.at[pl.ds(r0, RPT)], out.at[pl.ds(cid * NP + r0, RPT)])


@pl.kernel(
    out_type=[jax.ShapeDtypeStruct((NC * NP, D), jnp.float32)],
    mesh=_MESH,
    scratch_types=[
        pltpu.VMEM((CH,), jnp.int32),
        pltpu.VMEM((CH, D), jnp.float32),
        pltpu.VMEM_SHARED((NP, D), jnp.float32),
    ])
def _seg_count(dst, out, dst_v, ones_v, acc):
  cid = lax.axis_index("c")
  sid = lax.axis_index("s")
  wid = cid * NS + sid

  _zero_rows(ones_v, CH)
  r0 = sid * RPT
  for k in range(RPT // CH):
    pltpu.sync_copy(ones_v, acc.at[pl.ds(r0 + k * CH, CH)])

  def orow(i, carry):
    for j in range(D // 16):
      ones_v[i, pl.ds(j * 16, 16)] = jnp.ones((16,), jnp.float32)
    return carry
  lax.fori_loop(0, CH, orow, 0)
  plsc.subcore_barrier()

  ebase = wid * EPT
  def chunk(c, carry):
    base = pl.multiple_of(ebase + c * CH, 8)
    pltpu.sync_copy(dst.at[pl.ds(base, CH)], dst_v)
    pltpu.sync_copy(ones_v, acc.at[dst_v], add=True)
    return carry
  lax.fori_loop(0, NCHUNK, chunk, 0)
  plsc.subcore_barrier()

  pltpu.sync_copy(acc.at[pl.ds(r0, RPT)], out.at[pl.ds(cid * NP + r0, RPT)])


_BM = 1000


def _blk(r, c):
  return pl.BlockSpec((r, c), lambda i: (i, 0) if r == _BM else (0, 0))


def _tc_layer1(x, W_l, W_r, b):
  def body(x_ref, wl, wr, b_ref, y_ref, r_ref):
    xb = x_ref[...]
    y_ref[...] = jnp.dot(xb, wl[...], preferred_element_type=jnp.float32)
    r_ref[...] = jnp.dot(xb, wr[...], preferred_element_type=jnp.float32) + b_ref[...]
  return pl.pallas_call(
      body, grid=(N // _BM,),
      in_specs=[_blk(_BM, D), _blk(D, D), _blk(D, D), _blk(1, D)],
      out_specs=[_blk(_BM, D), _blk(_BM, D)],
      out_shape=[jax.ShapeDtypeStruct((N, D), jnp.float32)] * 2,
  )(x, W_l, W_r, b)


def _tc_mid(p0, p1, c0, c1, r1, W_l, W_r, b):
  def body(p0r, p1r, c0r, c1r, r1r, wl, wr, b_ref, y_ref, r_ref):
    s = p0r[...] + p1r[...]
    cnt = (c0r[...] + c1r[...])[:, 0:1]
    h = jnp.maximum(s / jnp.maximum(cnt, 1.0) + r1r[...], 0.0)
    y_ref[...] = jnp.dot(h, wl[...], preferred_element_type=jnp.float32)
    r_ref[...] = jnp.dot(h, wr[...], preferred_element_type=jnp.float32) + b_ref[...]
  return pl.pallas_call(
      body, grid=(N // _BM,),
      in_specs=[_blk(_BM, D), _blk(_BM, D), _blk(_BM, D), _blk(_BM, D),
                _blk(_BM, D), _blk(D, D), _blk(D, D), _blk(1, D)],
      out_specs=[_blk(_BM, D), _blk(_BM, D)],
      out_shape=[jax.ShapeDtypeStruct((N, D), jnp.float32)] * 2,
  )(p0, p1, c0, c1, r1, W_l, W_r, b)


def _tc_final(q0, q1, c0, c1, r2):
  def body(q0r, q1r, c0r, c1r, r2r, o_ref):
    s = q0r[...] + q1r[...]
    cnt = (c0r[...] + c1r[...])[:, 0:1]
    o_ref[...] = s / jnp.maximum(cnt, 1.0) + r2r[...]
  return pl.pallas_call(
      body, grid=(N // _BM,),
      in_specs=[_blk(_BM, D), _blk(_BM, D), _blk(_BM, D), _blk(_BM, D),
                _blk(_BM, D)],
      out_specs=_blk(_BM, D),
      out_shape=jax.ShapeDtypeStruct((N, D), jnp.float32),
  )(q0, q1, c0, c1, r2)


def kernel(x, edge_index, W1_l, b1_l, W1_r, W2_l, b2_l, W2_r):
  src = edge_index[0]
  dst = edge_index[1]
  cnt, = _seg_count(dst)
  c0, c1 = cnt[:N], cnt[NP:NP + N]
  y1, r1 = _tc_layer1(x, W1_l, W1_r, b1_l.reshape(1, D))
  p, = _seg_sum(y1, src, dst)
  y2, r2 = _tc_mid(p[:N], p[NP:NP + N], c0, c1, r1, W2_l, W2_r, b2_l.reshape(1, D))
  q, = _seg_sum(y2, src, dst)
  return _tc_final(q[:N], q[NP:NP + N], c0, c1, r2)


# speedup vs baseline: 9.7322x; 2.0840x over previous
import jax
import jax.numpy as jnp
from jax import lax
from jax.experimental import pallas as pl
from jax.experimental.pallas import tpu as pltpu
from jax.experimental.pallas import tpu_sc as plsc

N = 10000
E = 320000
D = 128
NC, NS = 2, 16
NW = NC * NS
EPT = E // NW
CH = 80
NCHUNK = EPT // CH
NP = 10240
RPT = NP // NS

_MESH = plsc.VectorSubcoreMesh(core_axis_name="c", subcore_axis_name="s",
                               num_cores=NC, num_subcores=NS)


def _zero_rows(buf, rows):
  def zrow(i, carry):
    for j in range(D // 16):
      buf[i, pl.ds(j * 16, 16)] = jnp.zeros((16,), jnp.float32)
    return carry
  lax.fori_loop(0, rows, zrow, 0)


@pl.kernel(
    out_type=[jax.ShapeDtypeStruct((NC * NP, D), jnp.float32)],
    mesh=_MESH,
    scratch_types=[
        pltpu.VMEM((EPT,), jnp.int32),
        pltpu.VMEM((NCHUNK, CH), jnp.int32),
        pltpu.VMEM((CH, D), jnp.float32),
        pltpu.VMEM((CH, D), jnp.float32),
        pltpu.VMEM_SHARED((NP, D), jnp.float32),
        pltpu.SemaphoreType.DMA,
        pltpu.SemaphoreType.DMA,
    ])
def _seg_sum(tbl, src, dst3, out, src_a, dst_a, rows0, rows1, acc, sem0, sem1):
  cid = lax.axis_index("c")
  sid = lax.axis_index("s")
  wid = cid * NS + sid

  _zero_rows(rows0, CH)
  r0 = sid * RPT
  for k in range(RPT // CH):
    pltpu.sync_copy(rows0, acc.at[pl.ds(r0 + k * CH, CH)])
  plsc.subcore_barrier()

  ebase = pl.multiple_of(wid * EPT, 8)
  pltpu.sync_copy(src.at[pl.ds(ebase, EPT)], src_a)
  pltpu.sync_copy(dst3.at[wid], dst_a)

  def gather_start(c, buf, sem):
    off = pl.multiple_of(c * CH, 8)
    pltpu.async_copy(tbl.at[src_a.at[pl.ds(off, CH)]], buf, sem)

  def gather_wait(c, buf, sem):
    off = pl.multiple_of(c * CH, 8)
    pltpu.make_async_copy(tbl.at[src_a.at[pl.ds(off, CH)]], buf, sem).wait()

  gather_start(0, rows0, sem0)
  def pipe(c2, carry):
    c = c2 * 2
    gather_start(c + 1, rows1, sem1)
    gather_wait(c, rows0, sem0)
    pltpu.sync_copy(rows0, acc.at[dst_a.at[c]], add=True)
    gather_start(c + 2, rows0, sem0)
    gather_wait(c + 1, rows1, sem1)
    pltpu.sync_copy(rows1, acc.at[dst_a.at[c + 1]], add=True)
    return carry
  lax.fori_loop(0, (NCHUNK - 1) // 2, pipe, 0)
  gather_wait(NCHUNK - 1, rows0, sem0)
  pltpu.sync_copy(rows0, acc.at[dst_a.at[NCHUNK - 1]], add=True)
  plsc.subcore_barrier()

  pltpu.sync_copy(acc.at[pl.ds(r0, RPT)], out.at[pl.ds(cid * NP + r0, RPT)])


@pl.kernel(
    out_type=[jax.ShapeDtypeStruct((NC * NP, D), jnp.float32)],
    mesh=_MESH,
    scratch_types=[
        pltpu.VMEM((NCHUNK, CH), jnp.int32),
        pltpu.VMEM((CH, D), jnp.float32),
        pltpu.VMEM_SHARED((NP, D), jnp.float32),
    ])
def _seg_count(dst3, out, dst_a, ones_v, acc):
  cid = lax.axis_index("c")
  sid = lax.axis_index("s")
  wid = cid * NS + sid

  _zero_rows(ones_v, CH)
  r0 = sid * RPT
  for k in range(RPT // CH):
    pltpu.sync_copy(ones_v, acc.at[pl.ds(r0 + k * CH, CH)])

  def orow(i, carry):
    for j in range(D // 16):
      ones_v[i, pl.ds(j * 16, 16)] = jnp.ones((16,), jnp.float32)
    return carry
  lax.fori_loop(0, CH, orow, 0)
  plsc.subcore_barrier()

  pltpu.sync_copy(dst3.at[wid], dst_a)
  def chunk(c, carry):
    pltpu.sync_copy(ones_v, acc.at[dst_a.at[c]], add=True)
    return carry
  lax.fori_loop(0, NCHUNK, chunk, 0)
  plsc.subcore_barrier()

  pltpu.sync_copy(acc.at[pl.ds(r0, RPT)], out.at[pl.ds(cid * NP + r0, RPT)])


_BM = 1000


def _blk(r, c):
  return pl.BlockSpec((r, c), lambda i: (i, 0) if r == _BM else (0, 0))


def _tc_layer1(x, W_l, W_r, b):
  def body(x_ref, wl, wr, b_ref, y_ref, r_ref):
    xb = x_ref[...]
    y_ref[...] = jnp.dot(xb, wl[...], preferred_element_type=jnp.float32)
    r_ref[...] = jnp.dot(xb, wr[...], preferred_element_type=jnp.float32) + b_ref[...]
  return pl.pallas_call(
      body, grid=(N // _BM,),
      in_specs=[_blk(_BM, D), _blk(D, D), _blk(D, D), _blk(1, D)],
      out_specs=[_blk(_BM, D), _blk(_BM, D)],
      out_shape=[jax.ShapeDtypeStruct((N, D), jnp.float32)] * 2,
  )(x, W_l, W_r, b)


def _tc_mid(p0, p1, c0, c1, r1, W_l, W_r, b):
  def body(p0r, p1r, c0r, c1r, r1r, wl, wr, b_ref, y_ref, r_ref):
    s = p0r[...] + p1r[...]
    cnt = (c0r[...] + c1r[...])[:, 0:1]
    h = jnp.maximum(s / jnp.maximum(cnt, 1.0) + r1r[...], 0.0)
    y_ref[...] = jnp.dot(h, wl[...], preferred_element_type=jnp.float32)
    r_ref[...] = jnp.dot(h, wr[...], preferred_element_type=jnp.float32) + b_ref[...]
  return pl.pallas_call(
      body, grid=(N // _BM,),
      in_specs=[_blk(_BM, D), _blk(_BM, D), _blk(_BM, D), _blk(_BM, D),
                _blk(_BM, D), _blk(D, D), _blk(D, D), _blk(1, D)],
      out_specs=[_blk(_BM, D), _blk(_BM, D)],
      out_shape=[jax.ShapeDtypeStruct((N, D), jnp.float32)] * 2,
  )(p0, p1, c0, c1, r1, W_l, W_r, b)


def _tc_final(q0, q1, c0, c1, r2):
  def body(q0r, q1r, c0r, c1r, r2r, o_ref):
    s = q0r[...] + q1r[...]
    cnt = (c0r[...] + c1r[...])[:, 0:1]
    o_ref[...] = s / jnp.maximum(cnt, 1.0) + r2r[...]
  return pl.pallas_call(
      body, grid=(N // _BM,),
      in_specs=[_blk(_BM, D), _blk(_BM, D), _blk(_BM, D), _blk(_BM, D),
                _blk(_BM, D)],
      out_specs=_blk(_BM, D),
      out_shape=jax.ShapeDtypeStruct((N, D), jnp.float32),
  )(q0, q1, c0, c1, r2)


def kernel(x, edge_index, W1_l, b1_l, W1_r, W2_l, b2_l, W2_r):
  src = edge_index[0]
  dst3 = edge_index[1].reshape(NW, NCHUNK, CH)
  cnt, = _seg_count(dst3)
  c0, c1 = cnt[:N], cnt[NP:NP + N]
  y1, r1 = _tc_layer1(x, W1_l, W1_r, b1_l.reshape(1, D))
  p, = _seg_sum(y1, src, dst3)
  y2, r2 = _tc_mid(p[:N], p[NP:NP + N], c0, c1, r1, W2_l, W2_r, b2_l.reshape(1, D))
  q, = _seg_sum(y2, src, dst3)
  return _tc_final(q[:N], q[NP:NP + N], c0, c1, r2)
